# Initial kernel scaffold; baseline (speedup 1.0000x reference)
import jax
import jax.numpy as jnp
from jax.experimental import pallas as pl


def kernel(x, edge_index, W1, a_src1, a_dst1, b1, W2, a_src2, a_dst2, b2):
    raise NotImplementedError("write your pallas kernel here")


# speedup vs baseline: 14.6497x; 14.6497x over previous
import functools

import jax
import jax.numpy as jnp
from jax import lax
from jax.experimental import pallas as pl
from jax.experimental.pallas import tpu as pltpu
from jax.experimental.pallas import tpu_sc as plsc

N = 10000
E = 320000
D_HID = 128
N_CLS = 40

N_PAD = 10240
NW = 32
E_PAD = 327680
EPW = E_PAD // NW
K = 128
NCHUNK = EPW // K
D = 128

ROWS_PER_TILE = N_PAD // 16
ZROWS = 64


def _linatt_body(x_ref, w_ref, v_ref, h_ref, al_ref):
    h = jnp.dot(x_ref[...], w_ref[...], preferred_element_type=jnp.float32)
    h_ref[...] = h
    al_ref[...] = jnp.dot(h, v_ref[...], preferred_element_type=jnp.float32)


def _tc_linatt(xp, W, V):
    m, d = xp.shape
    dh = W.shape[1]
    blk = 1024
    return pl.pallas_call(
        _linatt_body,
        grid=(m // blk,),
        in_specs=[
            pl.BlockSpec((blk, d), lambda i: (i, 0)),
            pl.BlockSpec((d, dh), lambda i: (0, 0)),
            pl.BlockSpec((dh, 2), lambda i: (0, 0)),
        ],
        out_specs=[
            pl.BlockSpec((blk, dh), lambda i: (i, 0)),
            pl.BlockSpec((blk, 2), lambda i: (i, 0)),
        ],
        out_shape=[
            jax.ShapeDtypeStruct((m, dh), jnp.float32),
            jax.ShapeDtypeStruct((m, 2), jnp.float32),
        ],
    )(xp, W, V)


def _mid_body(accA_ref, accB_ref, s32_ref, h_ref, al_ref, b_ref, w2_ref, a2_ref,
              h2_ref, al2_ref):
    e = al_ref[:, 0:1] + al_ref[:, 1:2]
    e = jnp.where(e >= 0.0, e, 0.2 * e)
    wself = jnp.exp(e)
    s = jnp.sum(s32_ref[...], axis=0)[:, None] + wself + 1e-16
    num = accA_ref[...] + accB_ref[...] + wself * h_ref[...]
    g = num / s + b_ref[...]
    h2pre = jnp.maximum(g, 0.0)
    h2 = jnp.dot(h2pre, w2_ref[...], preferred_element_type=jnp.float32)
    al2_ref[...] = jnp.dot(h2, a2_ref[...], preferred_element_type=jnp.float32)
    col = lax.broadcasted_iota(jnp.int32, h2.shape, 1)
    h2_ref[...] = jnp.where(col == N_CLS, 1.0, h2)


def _tc_mid(accA, accB, s32, h1, al1, b1r, W2e, A2e):
    m, d = accA.shape
    blk = 1024
    return pl.pallas_call(
        _mid_body,
        grid=(m // blk,),
        in_specs=[
            pl.BlockSpec((blk, d), lambda i: (i, 0)),
            pl.BlockSpec((blk, d), lambda i: (i, 0)),
            pl.BlockSpec((NW, blk), lambda i: (0, i)),
            pl.BlockSpec((blk, d), lambda i: (i, 0)),
            pl.BlockSpec((blk, 2), lambda i: (i, 0)),
            pl.BlockSpec((1, d), lambda i: (0, 0)),
            pl.BlockSpec((d, d), lambda i: (0, 0)),
            pl.BlockSpec((d, 2), lambda i: (0, 0)),
        ],
        out_specs=[
            pl.BlockSpec((blk, d), lambda i: (i, 0)),
            pl.BlockSpec((blk, 2), lambda i: (i, 0)),
        ],
        out_shape=[
            jax.ShapeDtypeStruct((m, d), jnp.float32),
            jax.ShapeDtypeStruct((m, 2), jnp.float32),
        ],
    )(accA, accB, s32, h1, al1, b1r, W2e, A2e)


def _final_body(accA_ref, accB_ref, h_ref, al_ref, b_ref, out_ref):
    c = N_CLS
    e = al_ref[:, 0:1] + al_ref[:, 1:2]
    e = jnp.where(e >= 0.0, e, 0.2 * e)
    wself = jnp.exp(e)
    s = accA_ref[:, c:c + 1] + accB_ref[:, c:c + 1] + wself + 1e-16
    num = accA_ref[:, :c] + accB_ref[:, :c] + wself * h_ref[:, :c]
    g = num / s + b_ref[...]
    m = jnp.max(g, axis=1, keepdims=True)
    z = g - m
    out_ref[...] = z - jnp.log(jnp.sum(jnp.exp(z), axis=1, keepdims=True))


def _tc_final(accA, accB, h2e, al2, b2r):
    m, d = accA.shape
    c = N_CLS
    blk = 1024
    return pl.pallas_call(
        _final_body,
        grid=(m // blk,),
        in_specs=[
            pl.BlockSpec((blk, d), lambda i: (i, 0)),
            pl.BlockSpec((blk, d), lambda i: (i, 0)),
            pl.BlockSpec((blk, d), lambda i: (i, 0)),
            pl.BlockSpec((blk, 2), lambda i: (i, 0)),
            pl.BlockSpec((1, c), lambda i: (0, 0)),
        ],
        out_specs=pl.BlockSpec((blk, c), lambda i: (i, 0)),
        out_shape=jax.ShapeDtypeStruct((m, c), jnp.float32),
    )(accA, accB, h2e, al2, b2r)


def _sc_edge(with_s):
    mesh = plsc.VectorSubcoreMesh(core_axis_name="c", subcore_axis_name="s")

    out_type = [jax.ShapeDtypeStruct((2, N_PAD, D), jnp.float32)]
    scratch = [
        pltpu.VMEM((N_PAD,), jnp.float32),
        pltpu.VMEM((N_PAD,), jnp.float32),
        pltpu.VMEM((K,), jnp.int32),
        pltpu.VMEM((K,), jnp.int32),
        pltpu.VMEM((K,), jnp.float32),
        pltpu.VMEM((K, D), jnp.float32),
        pltpu.VMEM_SHARED((N_PAD, D), jnp.float32),
        pltpu.SemaphoreType.DMA,
    ]
    if with_s:
        out_type.append(jax.ShapeDtypeStruct((NW, N_PAD), jnp.float32))
        scratch.insert(6, pltpu.VMEM((N_PAD,), jnp.float32))

    @functools.partial(
        pl.kernel,
        mesh=mesh,
        compiler_params=pltpu.CompilerParams(needs_layout_passes=False),
        out_type=out_type,
        scratch_types=scratch,
    )
    def k(src_hbm, dst_hbm, als_hbm, ald_hbm, h_hbm, *rest):
        if with_s:
            (out_hbm, s_hbm, als_v, ald_v, sidx, didx, wbuf, gbuf,
             s_local, acc, sem) = rest
        else:
            (out_hbm, als_v, ald_v, sidx, didx, wbuf, gbuf, acc, sem) = rest
        cid = lax.axis_index("c")
        sid = lax.axis_index("s")
        base = (cid * 16 + sid) * EPW
        zvec = jnp.zeros((16,), jnp.float32)

        pltpu.sync_copy(als_hbm, als_v)
        pltpu.sync_copy(ald_hbm, ald_v)

        def _zrow(i, _):
            for j in range(D // 16):
                gbuf[i, pl.ds(j * 16, 16)] = zvec
            return 0
        lax.fori_loop(0, K, _zrow, 0)
        for t in range(ROWS_PER_TILE // K):
            pltpu.sync_copy(gbuf, acc.at[pl.ds(sid * ROWS_PER_TILE + t * K, K)])

        if with_s:
            def _zs(i, _):
                s_local[pl.ds(i * 16, 16)] = zvec
                return 0
            lax.fori_loop(0, N_PAD // 16, _zs, 0)
        plsc.subcore_barrier()

        def _chunk(ci, _):
            off = base + ci * K
            pltpu.sync_copy(src_hbm.at[pl.ds(off, K)], sidx)
            pltpu.sync_copy(dst_hbm.at[pl.ds(off, K)], didx)
            gather = pltpu.async_copy(h_hbm.at[sidx], gbuf, sem)
            for j in range(K // 16):
                sv = sidx[pl.ds(j * 16, 16)]
                dv = didx[pl.ds(j * 16, 16)]
                a1 = plsc.load_gather(als_v, [sv])
                a2 = plsc.load_gather(ald_v, [dv])
                e = a1 + a2
                e = jnp.where(e >= 0.0, e, 0.2 * e)
                w = jnp.exp(e)
                wbuf[pl.ds(j * 16, 16)] = w
                if with_s:
                    plsc.addupdate_scatter(s_local, [dv], w)
            gather.wait()

            def _srow(r, _):
                wv = plsc.load_gather(wbuf, [jnp.full((16,), r, jnp.int32)])
                for q in range(D // 16):
                    gbuf[r, pl.ds(q * 16, 16)] = gbuf[r, pl.ds(q * 16, 16)] * wv
                return 0
            lax.fori_loop(0, K, _srow, 0)

            pltpu.sync_copy(gbuf, acc.at[didx], add=True)
            return 0
        lax.fori_loop(0, NCHUNK, _chunk, 0)

        plsc.subcore_barrier()
        pltpu.sync_copy(acc.at[pl.ds(sid * ROWS_PER_TILE, ROWS_PER_TILE)],
                        out_hbm.at[cid, pl.ds(sid * ROWS_PER_TILE, ROWS_PER_TILE)])
        if with_s:
            pltpu.sync_copy(s_local, s_hbm.at[cid * 16 + sid])

    return k


_sc_edge_l1 = _sc_edge(True)
_sc_edge_l2 = _sc_edge(False)


def kernel(x, edge_index, W1, a_src1, a_dst1, b1, W2, a_src2, a_dst2, b2):
    xp = jnp.pad(x, ((0, N_PAD - N), (0, 0)))
    npad = E_PAD - E
    srcp = jnp.concatenate([edge_index[0], jnp.zeros((npad,), jnp.int32)])
    dstp = jnp.concatenate(
        [edge_index[1], N + (jnp.arange(npad, dtype=jnp.int32) % (N_PAD - N))])

    V1 = jnp.stack([a_src1, a_dst1], axis=1)
    W2e = jnp.pad(W2, ((0, 0), (0, D - N_CLS)))
    A2e = jnp.pad(jnp.stack([a_src2, a_dst2], axis=1), ((0, D - N_CLS), (0, 0)))
    b1r = b1.reshape(1, -1)
    b2r = b2.reshape(1, -1)

    h1, al1 = _tc_linatt(xp, W1, V1)
    acc1, s1 = _sc_edge_l1(srcp, dstp, al1[:, 0], al1[:, 1], h1)
    h2e, al2 = _tc_mid(acc1[0], acc1[1], s1, h1, al1, b1r, W2e, A2e)
    acc2 = _sc_edge_l2(srcp, dstp, al2[:, 0], al2[:, 1], h2e)
    out = _tc_final(acc2[0][0], acc2[0][1], h2e, al2, b2r)
    return out[:N]


# speedup vs baseline: 17.6965x; 1.2080x over previous
import functools

import jax
import jax.numpy as jnp
from jax import lax
from jax.experimental import pallas as pl
from jax.experimental.pallas import tpu as pltpu
from jax.experimental.pallas import tpu_sc as plsc

N = 10000
E = 320000
D_HID = 128
N_CLS = 40

N_PAD = 10240
NW = 32
E_PAD = 327680
EPW = E_PAD // NW
K = 128
NCHUNK = EPW // K
D = 128

ROWS_PER_TILE = N_PAD // 16
ZROWS = 64


def _linatt_body(x_ref, w_ref, v_ref, h_ref, al_ref):
    h = jnp.dot(x_ref[...], w_ref[...], preferred_element_type=jnp.float32)
    h_ref[...] = h
    al_ref[...] = jnp.dot(h, v_ref[...], preferred_element_type=jnp.float32)


def _tc_linatt(xp, W, V):
    m, d = xp.shape
    dh = W.shape[1]
    blk = 1024
    return pl.pallas_call(
        _linatt_body,
        grid=(m // blk,),
        in_specs=[
            pl.BlockSpec((blk, d), lambda i: (i, 0)),
            pl.BlockSpec((d, dh), lambda i: (0, 0)),
            pl.BlockSpec((dh, 2), lambda i: (0, 0)),
        ],
        out_specs=[
            pl.BlockSpec((blk, dh), lambda i: (i, 0)),
            pl.BlockSpec((blk, 2), lambda i: (i, 0)),
        ],
        out_shape=[
            jax.ShapeDtypeStruct((m, dh), jnp.float32),
            jax.ShapeDtypeStruct((m, 2), jnp.float32),
        ],
    )(xp, W, V)


def _mid_body(accA_ref, accB_ref, s32_ref, h_ref, al_ref, b_ref, w2_ref, a2_ref,
              h2_ref, al2_ref):
    e = al_ref[:, 0:1] + al_ref[:, 1:2]
    e = jnp.where(e >= 0.0, e, 0.2 * e)
    wself = jnp.exp(e)
    s = jnp.sum(s32_ref[...], axis=0)[:, None] + wself + 1e-16
    num = accA_ref[...] + accB_ref[...] + wself * h_ref[...]
    g = num / s + b_ref[...]
    h2pre = jnp.maximum(g, 0.0)
    h2 = jnp.dot(h2pre, w2_ref[...], preferred_element_type=jnp.float32)
    al2_ref[...] = jnp.dot(h2, a2_ref[...], preferred_element_type=jnp.float32)
    col = lax.broadcasted_iota(jnp.int32, h2.shape, 1)
    h2_ref[...] = jnp.where(col == N_CLS, 1.0, h2)


def _tc_mid(accA, accB, s32, h1, al1, b1r, W2e, A2e):
    m, d = accA.shape
    blk = 1024
    return pl.pallas_call(
        _mid_body,
        grid=(m // blk,),
        in_specs=[
            pl.BlockSpec((blk, d), lambda i: (i, 0)),
            pl.BlockSpec((blk, d), lambda i: (i, 0)),
            pl.BlockSpec((NW, blk), lambda i: (0, i)),
            pl.BlockSpec((blk, d), lambda i: (i, 0)),
            pl.BlockSpec((blk, 2), lambda i: (i, 0)),
            pl.BlockSpec((1, d), lambda i: (0, 0)),
            pl.BlockSpec((d, d), lambda i: (0, 0)),
            pl.BlockSpec((d, 2), lambda i: (0, 0)),
        ],
        out_specs=[
            pl.BlockSpec((blk, d), lambda i: (i, 0)),
            pl.BlockSpec((blk, 2), lambda i: (i, 0)),
        ],
        out_shape=[
            jax.ShapeDtypeStruct((m, d), jnp.float32),
            jax.ShapeDtypeStruct((m, 2), jnp.float32),
        ],
    )(accA, accB, s32, h1, al1, b1r, W2e, A2e)


def _final_body(accA_ref, accB_ref, h_ref, al_ref, b_ref, out_ref):
    c = N_CLS
    e = al_ref[:, 0:1] + al_ref[:, 1:2]
    e = jnp.where(e >= 0.0, e, 0.2 * e)
    wself = jnp.exp(e)
    s = accA_ref[:, c:c + 1] + accB_ref[:, c:c + 1] + wself + 1e-16
    num = accA_ref[:, :c] + accB_ref[:, :c] + wself * h_ref[:, :c]
    g = num / s + b_ref[...]
    m = jnp.max(g, axis=1, keepdims=True)
    z = g - m
    out_ref[...] = z - jnp.log(jnp.sum(jnp.exp(z), axis=1, keepdims=True))


def _tc_final(accA, accB, h2e, al2, b2r):
    m, d = accA.shape
    c = N_CLS
    blk = 1024
    return pl.pallas_call(
        _final_body,
        grid=(m // blk,),
        in_specs=[
            pl.BlockSpec((blk, d), lambda i: (i, 0)),
            pl.BlockSpec((blk, d), lambda i: (i, 0)),
            pl.BlockSpec((blk, d), lambda i: (i, 0)),
            pl.BlockSpec((blk, 2), lambda i: (i, 0)),
            pl.BlockSpec((1, c), lambda i: (0, 0)),
        ],
        out_specs=pl.BlockSpec((blk, c), lambda i: (i, 0)),
        out_shape=jax.ShapeDtypeStruct((m, c), jnp.float32),
    )(accA, accB, h2e, al2, b2r)


KW = 1024
NWCHUNK = EPW // KW


def _sc_weights(with_s):
    mesh = plsc.VectorSubcoreMesh(core_axis_name="c", subcore_axis_name="s")

    out_type = [jax.ShapeDtypeStruct((E_PAD,), jnp.float32)]
    scratch = [
        pltpu.VMEM((N_PAD,), jnp.float32),
        pltpu.VMEM((N_PAD,), jnp.float32),
        pltpu.VMEM((KW,), jnp.int32),
        pltpu.VMEM((KW,), jnp.int32),
        pltpu.VMEM((KW,), jnp.int32),
        pltpu.VMEM((KW,), jnp.int32),
        pltpu.VMEM((KW,), jnp.float32),
        pltpu.VMEM((KW,), jnp.float32),
        pltpu.SemaphoreType.DMA,
        pltpu.SemaphoreType.DMA,
        pltpu.SemaphoreType.DMA,
        pltpu.SemaphoreType.DMA,
    ]
    if with_s:
        out_type.append(jax.ShapeDtypeStruct((NW, N_PAD), jnp.float32))
        scratch.insert(8, pltpu.VMEM((N_PAD,), jnp.float32))

    @functools.partial(
        pl.kernel,
        mesh=mesh,
        compiler_params=pltpu.CompilerParams(needs_layout_passes=False),
        out_type=out_type,
        scratch_types=scratch,
    )
    def k(src_hbm, dst_hbm, als_hbm, ald_hbm, *rest):
        if with_s:
            (w_hbm, s_hbm, als_v, ald_v, sidxA, sidxB, didxA, didxB,
             wstA, wstB, s_local, isemA, isemB, wsemA, wsemB) = rest
        else:
            (w_hbm, als_v, ald_v, sidxA, sidxB, didxA, didxB,
             wstA, wstB, isemA, isemB, wsemA, wsemB) = rest
        cid = lax.axis_index("c")
        sid = lax.axis_index("s")
        base = (cid * 16 + sid) * EPW
        zvec = jnp.zeros((16,), jnp.float32)

        pltpu.sync_copy(als_hbm, als_v)
        pltpu.sync_copy(ald_hbm, ald_v)
        if with_s:
            def _zs(i, _):
                s_local[pl.ds(i * 16, 16)] = zvec
                return 0
            lax.fori_loop(0, N_PAD // 16, _zs, 0)

        def _drain_idx(sem):
            pltpu.make_async_copy(src_hbm.at[pl.ds(0, KW)], sidxA, sem).wait()

        def _drain_w(sem):
            pltpu.make_async_copy(w_hbm.at[pl.ds(0, KW)], wstA, sem).wait()

        def _issue_idx(i, sidx, didx, isem):
            off = base + jnp.minimum(i, NWCHUNK - 1) * KW
            pltpu.async_copy(src_hbm.at[pl.ds(off, KW)], sidx, isem)
            pltpu.async_copy(dst_hbm.at[pl.ds(off, KW)], didx, isem)

        def _compute(i, sidx, didx, wst, wsem):
            def _grp(j, _):
                sv = sidx[pl.ds(j * 16, 16)]
                dv = didx[pl.ds(j * 16, 16)]
                e = plsc.load_gather(als_v, [sv]) + plsc.load_gather(ald_v, [dv])
                e = jnp.where(e >= 0.0, e, 0.2 * e)
                w = jnp.exp(e)
                wst[pl.ds(j * 16, 16)] = w
                if with_s:
                    plsc.addupdate_scatter(s_local, [dv], w)
                return 0
            lax.fori_loop(0, KW // 16, _grp, 0)
            pltpu.async_copy(wst, w_hbm.at[pl.ds(base + i * KW, KW)], wsem)

        def _half(i, sidxX, didxX, isemX, wsemX, sidxY, didxY, isemY, wstY, wsemY):
            @pl.when(i > 1)
            def _():
                _drain_w(wsemX)
            _issue_idx(i, sidxX, didxX, isemX)
            _drain_idx(isemY)
            _drain_idx(isemY)
            _compute(i - 1, sidxY, didxY, wstY, wsemY)

        _issue_idx(0, sidxA, didxA, isemA)

        def _pair(tt, _):
            i = 2 * tt + 1
            _half(i, sidxB, didxB, isemB, wsemB, sidxA, didxA, isemA, wstA, wsemA)
            _half(i + 1, sidxA, didxA, isemA, wsemA, sidxB, didxB, isemB, wstB, wsemB)
            return 0
        lax.fori_loop(0, NWCHUNK // 2, _pair, 0)

        _drain_idx(isemA)
        _drain_idx(isemA)
        _drain_w(wsemB)
        if with_s:
            pltpu.sync_copy(s_local, s_hbm.at[cid * 16 + sid])

    return k


def _sc_rows():
    mesh = plsc.VectorSubcoreMesh(core_axis_name="c", subcore_axis_name="s")

    @functools.partial(
        pl.kernel,
        mesh=mesh,
        compiler_params=pltpu.CompilerParams(needs_layout_passes=False),
        out_type=jax.ShapeDtypeStruct((2, N_PAD, D), jnp.float32),
        scratch_types=[
            pltpu.VMEM((K,), jnp.int32),
            pltpu.VMEM((K,), jnp.int32),
            pltpu.VMEM((K,), jnp.int32),
            pltpu.VMEM((K,), jnp.int32),
            pltpu.VMEM((K,), jnp.float32),
            pltpu.VMEM((K,), jnp.float32),
            pltpu.VMEM((K, D), jnp.float32),
            pltpu.VMEM((K, D), jnp.float32),
            pltpu.VMEM_SHARED((N_PAD, D), jnp.float32),
            pltpu.SemaphoreType.DMA,
            pltpu.SemaphoreType.DMA,
            pltpu.SemaphoreType.DMA,
            pltpu.SemaphoreType.DMA,
            pltpu.SemaphoreType.DMA,
            pltpu.SemaphoreType.DMA,
        ],
    )
    def k(src_hbm, dst_hbm, w_hbm, h_hbm, out_hbm,
          sidxA, sidxB, didxA, didxB, wbufA, wbufB, gbufA, gbufB, acc,
          isemA, isemB, gsemA, gsemB, ssemA, ssemB):
        cid = lax.axis_index("c")
        sid = lax.axis_index("s")
        base = (cid * 16 + sid) * EPW
        zvec = jnp.zeros((16,), jnp.float32)

        def _zrow(i, _):
            for j in range(D // 16):
                gbufA[i, pl.ds(j * 16, 16)] = zvec
            return 0
        lax.fori_loop(0, K, _zrow, 0)
        for t in range(ROWS_PER_TILE // K):
            pltpu.sync_copy(gbufA, acc.at[pl.ds(sid * ROWS_PER_TILE + t * K, K)])
        plsc.subcore_barrier()

        def _drain_small(sem):
            pltpu.make_async_copy(src_hbm.at[pl.ds(0, K)], sidxA, sem).wait()

        def _drain_rows(sem):
            pltpu.make_async_copy(h_hbm.at[pl.ds(0, K)], gbufA, sem).wait()

        def _issue_idx(i, sidx, didx, wbuf, isem):
            off = base + jnp.minimum(i, NCHUNK - 1) * K
            pltpu.async_copy(src_hbm.at[pl.ds(off, K)], sidx, isem)
            pltpu.async_copy(dst_hbm.at[pl.ds(off, K)], didx, isem)
            pltpu.async_copy(w_hbm.at[pl.ds(off, K)], wbuf, isem)

        def _scale(gbuf, wbuf):
            def _srow(r, _):
                wv = plsc.load_gather(wbuf, [jnp.full((16,), r, jnp.int32)])
                for q in range(D // 16):
                    gbuf[r, pl.ds(q * 16, 16)] = gbuf[r, pl.ds(q * 16, 16)] * wv
                return 0
            lax.fori_loop(0, K, _srow, 0)

        def _half(i, sidxX, didxX, wbufX, gbufX, isemX, gsemX, ssemX,
                  didxY, wbufY, gbufY, gsemY, ssemY):
            @pl.when(i > 1)
            def _():
                _drain_rows(ssemX)
            _issue_idx(i, sidxX, didxX, wbufX, isemX)
            _drain_rows(gsemY)
            _scale(gbufY, wbufY)
            for _ in range(3):
                _drain_small(isemX)
            pltpu.async_copy(h_hbm.at[sidxX], gbufX, gsemX)
            pltpu.async_copy(gbufY, acc.at[didxY], ssemY, add=True)

        _issue_idx(0, sidxA, didxA, wbufA, isemA)
        for _ in range(3):
            _drain_small(isemA)
        pltpu.async_copy(h_hbm.at[sidxA], gbufA, gsemA)

        def _pair(tt, _):
            i = 2 * tt + 1
            _half(i, sidxB, didxB, wbufB, gbufB, isemB, gsemB, ssemB,
                  didxA, wbufA, gbufA, gsemA, ssemA)
            _half(i + 1, sidxA, didxA, wbufA, gbufA, isemA, gsemA, ssemA,
                  didxB, wbufB, gbufB, gsemB, ssemB)
            return 0
        lax.fori_loop(0, NCHUNK // 2, _pair, 0)

        _drain_rows(ssemB)
        _drain_rows(gsemA)

        plsc.subcore_barrier()
        pltpu.sync_copy(acc.at[pl.ds(sid * ROWS_PER_TILE, ROWS_PER_TILE)],
                        out_hbm.at[cid, pl.ds(sid * ROWS_PER_TILE, ROWS_PER_TILE)])

    return k


_sc_w_l1 = _sc_weights(True)
_sc_w_l2 = _sc_weights(False)
_sc_rows_k = _sc_rows()


def kernel(x, edge_index, W1, a_src1, a_dst1, b1, W2, a_src2, a_dst2, b2):
    xp = jnp.pad(x, ((0, N_PAD - N), (0, 0)))
    npad = E_PAD - E
    srcp = jnp.concatenate([edge_index[0], jnp.zeros((npad,), jnp.int32)])
    dstp = jnp.concatenate(
        [edge_index[1], N + (jnp.arange(npad, dtype=jnp.int32) % (N_PAD - N))])

    V1 = jnp.stack([a_src1, a_dst1], axis=1)
    W2e = jnp.pad(W2, ((0, 0), (0, D - N_CLS)))
    A2e = jnp.pad(jnp.stack([a_src2, a_dst2], axis=1), ((0, D - N_CLS), (0, 0)))
    b1r = b1.reshape(1, -1)
    b2r = b2.reshape(1, -1)

    h1, al1 = _tc_linatt(xp, W1, V1)
    w1, s1 = _sc_w_l1(srcp, dstp, al1[:, 0], al1[:, 1])
    acc1 = _sc_rows_k(srcp, dstp, w1, h1)
    h2e, al2 = _tc_mid(acc1[0], acc1[1], s1, h1, al1, b1r, W2e, A2e)
    w2 = _sc_w_l2(srcp, dstp, al2[:, 0], al2[:, 1])
    acc2 = _sc_rows_k(srcp, dstp, w2[0], h2e)
    out = _tc_final(acc2[0], acc2[1], h2e, al2, b2r)
    return out[:N]


# speedup vs baseline: 35.9513x; 2.0315x over previous
import functools

import jax
import jax.numpy as jnp
from jax import lax
from jax.experimental import pallas as pl
from jax.experimental.pallas import tpu as pltpu
from jax.experimental.pallas import tpu_sc as plsc

N = 10000
E = 320000
D_HID = 128
N_CLS = 40

N_PAD = 10240
NW = 32
E_PAD = 327680
EPW = E_PAD // NW
K = 128
NCHUNK = EPW // K
D = 128

ROWS_PER_TILE = N_PAD // 16
ZROWS = 64


def _linatt_body(x_ref, w_ref, v_ref, h_ref, al_ref):
    h = jnp.dot(x_ref[...], w_ref[...], preferred_element_type=jnp.float32)
    h_ref[...] = h
    al_ref[...] = jnp.dot(h, v_ref[...], preferred_element_type=jnp.float32)


def _tc_linatt(xp, W, V):
    m, d = xp.shape
    dh = W.shape[1]
    blk = 1024
    return pl.pallas_call(
        _linatt_body,
        grid=(m // blk,),
        in_specs=[
            pl.BlockSpec((blk, d), lambda i: (i, 0)),
            pl.BlockSpec((d, dh), lambda i: (0, 0)),
            pl.BlockSpec((dh, 2), lambda i: (0, 0)),
        ],
        out_specs=[
            pl.BlockSpec((blk, dh), lambda i: (i, 0)),
            pl.BlockSpec((blk, 2), lambda i: (i, 0)),
        ],
        out_shape=[
            jax.ShapeDtypeStruct((m, dh), jnp.float32),
            jax.ShapeDtypeStruct((m, 2), jnp.float32),
        ],
    )(xp, W, V)


def _mid_body(accA_ref, accB_ref, s32_ref, h_ref, al_ref, b_ref, w2_ref, a2_ref,
              h2_ref, al2_ref):
    e = al_ref[:, 0:1] + al_ref[:, 1:2]
    e = jnp.where(e >= 0.0, e, 0.2 * e)
    wself = jnp.exp(e)
    s = jnp.sum(s32_ref[...], axis=0)[:, None] + wself + 1e-16
    num = accA_ref[...] + accB_ref[...] + wself * h_ref[...]
    g = num / s + b_ref[...]
    h2pre = jnp.maximum(g, 0.0)
    h2 = jnp.dot(h2pre, w2_ref[...], preferred_element_type=jnp.float32)
    al2_ref[...] = jnp.dot(h2, a2_ref[...], preferred_element_type=jnp.float32)
    col = lax.broadcasted_iota(jnp.int32, h2.shape, 1)
    h2_ref[...] = jnp.where(col == N_CLS, 1.0, h2)


def _tc_mid(accA, accB, s32, h1, al1, b1r, W2e, A2e):
    m, d = accA.shape
    blk = 1024
    return pl.pallas_call(
        _mid_body,
        grid=(m // blk,),
        in_specs=[
            pl.BlockSpec((blk, d), lambda i: (i, 0)),
            pl.BlockSpec((blk, d), lambda i: (i, 0)),
            pl.BlockSpec((NW, blk), lambda i: (0, i)),
            pl.BlockSpec((blk, d), lambda i: (i, 0)),
            pl.BlockSpec((blk, 2), lambda i: (i, 0)),
            pl.BlockSpec((1, d), lambda i: (0, 0)),
            pl.BlockSpec((d, d), lambda i: (0, 0)),
            pl.BlockSpec((d, 2), lambda i: (0, 0)),
        ],
        out_specs=[
            pl.BlockSpec((blk, d), lambda i: (i, 0)),
            pl.BlockSpec((blk, 2), lambda i: (i, 0)),
        ],
        out_shape=[
            jax.ShapeDtypeStruct((m, d), jnp.float32),
            jax.ShapeDtypeStruct((m, 2), jnp.float32),
        ],
    )(accA, accB, s32, h1, al1, b1r, W2e, A2e)


def _final_body(accA_ref, accB_ref, h_ref, al_ref, b_ref, out_ref):
    c = N_CLS
    e = al_ref[:, 0:1] + al_ref[:, 1:2]
    e = jnp.where(e >= 0.0, e, 0.2 * e)
    wself = jnp.exp(e)
    s = accA_ref[:, c:c + 1] + accB_ref[:, c:c + 1] + wself + 1e-16
    num = accA_ref[:, :c] + accB_ref[:, :c] + wself * h_ref[:, :c]
    g = num / s + b_ref[...]
    m = jnp.max(g, axis=1, keepdims=True)
    z = g - m
    out_ref[...] = z - jnp.log(jnp.sum(jnp.exp(z), axis=1, keepdims=True))


def _tc_final(accA, accB, h2e, al2, b2r):
    m, d = accA.shape
    c = N_CLS
    blk = 1024
    return pl.pallas_call(
        _final_body,
        grid=(m // blk,),
        in_specs=[
            pl.BlockSpec((blk, d), lambda i: (i, 0)),
            pl.BlockSpec((blk, d), lambda i: (i, 0)),
            pl.BlockSpec((blk, d), lambda i: (i, 0)),
            pl.BlockSpec((blk, 2), lambda i: (i, 0)),
            pl.BlockSpec((1, c), lambda i: (0, 0)),
        ],
        out_specs=pl.BlockSpec((blk, c), lambda i: (i, 0)),
        out_shape=jax.ShapeDtypeStruct((m, c), jnp.float32),
    )(accA, accB, h2e, al2, b2r)


KW = 1024
NWCHUNK = EPW // KW


def _sc_weights(with_s):
    mesh = plsc.VectorSubcoreMesh(core_axis_name="c", subcore_axis_name="s")

    out_type = [jax.ShapeDtypeStruct((E_PAD,), jnp.float32)]
    scratch = [
        pltpu.VMEM((N_PAD,), jnp.float32),
        pltpu.VMEM((N_PAD,), jnp.float32),
        pltpu.VMEM((KW,), jnp.int32),
        pltpu.VMEM((KW,), jnp.int32),
        pltpu.VMEM((KW,), jnp.int32),
        pltpu.VMEM((KW,), jnp.int32),
        pltpu.VMEM((KW,), jnp.float32),
        pltpu.VMEM((KW,), jnp.float32),
        pltpu.SemaphoreType.DMA,
        pltpu.SemaphoreType.DMA,
        pltpu.SemaphoreType.DMA,
        pltpu.SemaphoreType.DMA,
    ]
    if with_s:
        out_type.append(jax.ShapeDtypeStruct((NW, N_PAD), jnp.float32))
        scratch.insert(8, pltpu.VMEM((N_PAD,), jnp.float32))

    @functools.partial(
        pl.kernel,
        mesh=mesh,
        compiler_params=pltpu.CompilerParams(needs_layout_passes=False),
        out_type=out_type,
        scratch_types=scratch,
    )
    def k(src_hbm, dst_hbm, als_hbm, ald_hbm, *rest):
        if with_s:
            (w_hbm, s_hbm, als_v, ald_v, sidxA, sidxB, didxA, didxB,
             wstA, wstB, s_local, isemA, isemB, wsemA, wsemB) = rest
        else:
            (w_hbm, als_v, ald_v, sidxA, sidxB, didxA, didxB,
             wstA, wstB, isemA, isemB, wsemA, wsemB) = rest
        cid = lax.axis_index("c")
        sid = lax.axis_index("s")
        base = (cid * 16 + sid) * EPW
        zvec = jnp.zeros((16,), jnp.float32)

        pltpu.sync_copy(als_hbm, als_v)
        pltpu.sync_copy(ald_hbm, ald_v)
        if with_s:
            def _zs(i, _):
                s_local[pl.ds(i * 16, 16)] = zvec
                return 0
            lax.fori_loop(0, N_PAD // 16, _zs, 0)

        def _drain_idx(sem):
            pltpu.make_async_copy(src_hbm.at[pl.ds(0, KW)], sidxA, sem).wait()

        def _drain_w(sem):
            pltpu.make_async_copy(w_hbm.at[pl.ds(0, KW)], wstA, sem).wait()

        def _issue_idx(i, sidx, didx, isem):
            off = base + jnp.minimum(i, NWCHUNK - 1) * KW
            pltpu.async_copy(src_hbm.at[pl.ds(off, KW)], sidx, isem)
            pltpu.async_copy(dst_hbm.at[pl.ds(off, KW)], didx, isem)

        def _compute(i, sidx, didx, wst, wsem):
            def _grp(j, _):
                sv = sidx[pl.ds(j * 16, 16)]
                dv = didx[pl.ds(j * 16, 16)]
                e = plsc.load_gather(als_v, [sv]) + plsc.load_gather(ald_v, [dv])
                e = jnp.where(e >= 0.0, e, 0.2 * e)
                w = jnp.exp(e)
                wst[pl.ds(j * 16, 16)] = w
                if with_s:
                    plsc.addupdate_scatter(s_local, [dv], w)
                return 0
            lax.fori_loop(0, KW // 16, _grp, 0)
            pltpu.async_copy(wst, w_hbm.at[pl.ds(base + i * KW, KW)], wsem)

        def _half(i, sidxX, didxX, isemX, wsemX, sidxY, didxY, isemY, wstY, wsemY):
            @pl.when(i > 1)
            def _():
                _drain_w(wsemX)
            _issue_idx(i, sidxX, didxX, isemX)
            _drain_idx(isemY)
            _drain_idx(isemY)
            _compute(i - 1, sidxY, didxY, wstY, wsemY)

        _issue_idx(0, sidxA, didxA, isemA)

        def _pair(tt, _):
            i = 2 * tt + 1
            _half(i, sidxB, didxB, isemB, wsemB, sidxA, didxA, isemA, wstA, wsemA)
            _half(i + 1, sidxA, didxA, isemA, wsemA, sidxB, didxB, isemB, wstB, wsemB)
            return 0
        lax.fori_loop(0, NWCHUNK // 2, _pair, 0)

        _drain_idx(isemA)
        _drain_idx(isemA)
        _drain_w(wsemB)
        if with_s:
            pltpu.sync_copy(s_local, s_hbm.at[cid * 16 + sid])

    return k


def _sc_rows():
    mesh = plsc.VectorSubcoreMesh(core_axis_name="c", subcore_axis_name="s")

    @functools.partial(
        pl.kernel,
        mesh=mesh,
        compiler_params=pltpu.CompilerParams(needs_layout_passes=False),
        out_type=jax.ShapeDtypeStruct((2, N_PAD, D), jnp.float32),
        scratch_types=[
            pltpu.VMEM((K,), jnp.int32),
            pltpu.VMEM((K,), jnp.int32),
            pltpu.VMEM((K,), jnp.int32),
            pltpu.VMEM((K,), jnp.int32),
            pltpu.VMEM((K,), jnp.float32),
            pltpu.VMEM((K,), jnp.float32),
            pltpu.VMEM((K, D), jnp.float32),
            pltpu.VMEM((K, D), jnp.float32),
            pltpu.VMEM_SHARED((N_PAD, D), jnp.float32),
            pltpu.SemaphoreType.DMA,
            pltpu.SemaphoreType.DMA,
            pltpu.SemaphoreType.DMA,
            pltpu.SemaphoreType.DMA,
            pltpu.SemaphoreType.DMA,
            pltpu.SemaphoreType.DMA,
        ],
    )
    def k(src_hbm, dst_hbm, w_hbm, h_hbm, out_hbm,
          sidxA, sidxB, didxA, didxB, wbufA, wbufB, gbufA, gbufB, acc,
          isemA, isemB, gsemA, gsemB, ssemA, ssemB):
        cid = lax.axis_index("c")
        sid = lax.axis_index("s")
        base = (cid * 16 + sid) * EPW
        zvec = jnp.zeros((16,), jnp.float32)

        def _zrow(i, _):
            for j in range(D // 16):
                gbufA[i, pl.ds(j * 16, 16)] = zvec
            return 0
        lax.fori_loop(0, K, _zrow, 0)
        for t in range(ROWS_PER_TILE // K):
            pltpu.sync_copy(gbufA, acc.at[pl.ds(sid * ROWS_PER_TILE + t * K, K)])
        plsc.subcore_barrier()

        def _drain_small(sem):
            pltpu.make_async_copy(src_hbm.at[pl.ds(0, K)], sidxA, sem).wait()

        def _drain_rows(sem):
            pltpu.make_async_copy(h_hbm.at[pl.ds(0, K)], gbufA, sem).wait()

        def _issue_idx(i, sidx, didx, wbuf, isem):
            off = base + jnp.minimum(i, NCHUNK - 1) * K
            pltpu.async_copy(src_hbm.at[pl.ds(off, K)], sidx, isem)
            pltpu.async_copy(dst_hbm.at[pl.ds(off, K)], didx, isem)
            pltpu.async_copy(w_hbm.at[pl.ds(off, K)], wbuf, isem)

        def _scale(gbuf, wbuf):
            def _srow(r, _):
                wv = plsc.load_gather(wbuf, [jnp.full((16,), r, jnp.int32)])
                for q in range(D // 16):
                    gbuf[r, pl.ds(q * 16, 16)] = gbuf[r, pl.ds(q * 16, 16)] * wv
                return 0
            lax.fori_loop(0, K, _srow, 0)

        def _half(i, sidxX, didxX, wbufX, gbufX, isemX, gsemX, ssemX,
                  didxY, wbufY, gbufY, gsemY, ssemY):
            @pl.when(i > 1)
            def _():
                _drain_rows(ssemX)
            _issue_idx(i, sidxX, didxX, wbufX, isemX)
            _drain_rows(gsemY)
            _scale(gbufY, wbufY)
            for _ in range(3):
                _drain_small(isemX)
            pltpu.async_copy(h_hbm.at[sidxX], gbufX, gsemX)
            pltpu.async_copy(gbufY, acc.at[didxY], ssemY, add=True)

        _issue_idx(0, sidxA, didxA, wbufA, isemA)
        for _ in range(3):
            _drain_small(isemA)
        pltpu.async_copy(h_hbm.at[sidxA], gbufA, gsemA)

        def _pair(tt, _):
            i = 2 * tt + 1
            _half(i, sidxB, didxB, wbufB, gbufB, isemB, gsemB, ssemB,
                  didxA, wbufA, gbufA, gsemA, ssemA)
            _half(i + 1, sidxA, didxA, wbufA, gbufA, isemA, gsemA, ssemA,
                  didxB, wbufB, gbufB, gsemB, ssemB)
            return 0
        lax.fori_loop(0, NCHUNK // 2, _pair, 0)

        _drain_rows(ssemB)
        _drain_rows(gsemA)

        plsc.subcore_barrier()
        pltpu.sync_copy(acc.at[pl.ds(sid * ROWS_PER_TILE, ROWS_PER_TILE)],
                        out_hbm.at[cid, pl.ds(sid * ROWS_PER_TILE, ROWS_PER_TILE)])

    return k


_sc_w_l1 = _sc_weights(True)
_sc_w_l2 = _sc_weights(False)
_sc_rows_k = _sc_rows()


def kernel(x, edge_index, W1, a_src1, a_dst1, b1, W2, a_src2, a_dst2, b2):
    xp = jnp.pad(x, ((0, N_PAD - N), (0, 0)))
    npad = E_PAD - E
    srcp = jnp.concatenate(
        [edge_index[0], jnp.arange(npad, dtype=jnp.int32) * 997 % N])
    dstp = jnp.concatenate(
        [edge_index[1], N + (jnp.arange(npad, dtype=jnp.int32) % (N_PAD - N))])

    V1 = jnp.stack([a_src1, a_dst1], axis=1)
    W2e = jnp.pad(W2, ((0, 0), (0, D - N_CLS)))
    A2e = jnp.pad(jnp.stack([a_src2, a_dst2], axis=1), ((0, D - N_CLS), (0, 0)))
    b1r = b1.reshape(1, -1)
    b2r = b2.reshape(1, -1)

    h1, al1 = _tc_linatt(xp, W1, V1)
    w1, s1 = _sc_w_l1(srcp, dstp, al1[:, 0], al1[:, 1])
    acc1 = _sc_rows_k(srcp, dstp, w1, h1)
    h2e, al2 = _tc_mid(acc1[0], acc1[1], s1, h1, al1, b1r, W2e, A2e)
    w2 = _sc_w_l2(srcp, dstp, al2[:, 0], al2[:, 1])
    acc2 = _sc_rows_k(srcp, dstp, w2[0], h2e)
    out = _tc_final(acc2[0], acc2[1], h2e, al2, b2r)
    return out[:N]


# speedup vs baseline: 42.1558x; 1.1726x over previous
import functools

import jax
import jax.numpy as jnp
from jax import lax
from jax.experimental import pallas as pl
from jax.experimental.pallas import tpu as pltpu
from jax.experimental.pallas import tpu_sc as plsc

N = 10000
E = 320000
D_HID = 128
N_CLS = 40

N_PAD = 10240
NW = 32
E_PAD = 327680
EPW = E_PAD // NW
KR = 80
NRCHUNK = EPW // KR
D = 128

ROWS_PER_TILE = N_PAD // 16


def _linatt_body(x_ref, w_ref, v_ref, h_ref, al_ref):
    h = jnp.dot(x_ref[...], w_ref[...], preferred_element_type=jnp.float32)
    h_ref[...] = h
    al_ref[...] = jnp.dot(h, v_ref[...], preferred_element_type=jnp.float32)


def _tc_linatt(xp, W, V):
    m, d = xp.shape
    dh = W.shape[1]
    blk = 1024
    return pl.pallas_call(
        _linatt_body,
        grid=(m // blk,),
        in_specs=[
            pl.BlockSpec((blk, d), lambda i: (i, 0)),
            pl.BlockSpec((d, dh), lambda i: (0, 0)),
            pl.BlockSpec((dh, 2), lambda i: (0, 0)),
        ],
        out_specs=[
            pl.BlockSpec((blk, dh), lambda i: (i, 0)),
            pl.BlockSpec((blk, 2), lambda i: (i, 0)),
        ],
        out_shape=[
            jax.ShapeDtypeStruct((m, dh), jnp.float32),
            jax.ShapeDtypeStruct((m, 2), jnp.float32),
        ],
    )(xp, W, V)


def _mid_body(accA_ref, accB_ref, s32_ref, h_ref, al_ref, b_ref, w2_ref, a2_ref,
              h2_ref, al2_ref):
    e = al_ref[:, 0:1] + al_ref[:, 1:2]
    e = jnp.where(e >= 0.0, e, 0.2 * e)
    wself = jnp.exp(e)
    s = jnp.sum(s32_ref[...], axis=0)[:, None] + wself + 1e-16
    num = accA_ref[...] + accB_ref[...] + wself * h_ref[...]
    g = num / s + b_ref[...]
    h2pre = jnp.maximum(g, 0.0)
    h2 = jnp.dot(h2pre, w2_ref[...], preferred_element_type=jnp.float32)
    al2_ref[...] = jnp.dot(h2, a2_ref[...], preferred_element_type=jnp.float32)
    col = lax.broadcasted_iota(jnp.int32, h2.shape, 1)
    h2_ref[...] = jnp.where(col == N_CLS, 1.0, h2)


def _tc_mid(accA, accB, s32, h1, al1, b1r, W2e, A2e):
    m, d = accA.shape
    blk = 1024
    return pl.pallas_call(
        _mid_body,
        grid=(m // blk,),
        in_specs=[
            pl.BlockSpec((blk, d), lambda i: (i, 0)),
            pl.BlockSpec((blk, d), lambda i: (i, 0)),
            pl.BlockSpec((NW, blk), lambda i: (0, i)),
            pl.BlockSpec((blk, d), lambda i: (i, 0)),
            pl.BlockSpec((blk, 2), lambda i: (i, 0)),
            pl.BlockSpec((1, d), lambda i: (0, 0)),
            pl.BlockSpec((d, d), lambda i: (0, 0)),
            pl.BlockSpec((d, 2), lambda i: (0, 0)),
        ],
        out_specs=[
            pl.BlockSpec((blk, d), lambda i: (i, 0)),
            pl.BlockSpec((blk, 2), lambda i: (i, 0)),
        ],
        out_shape=[
            jax.ShapeDtypeStruct((m, d), jnp.float32),
            jax.ShapeDtypeStruct((m, 2), jnp.float32),
        ],
    )(accA, accB, s32, h1, al1, b1r, W2e, A2e)


def _final_body(accA_ref, accB_ref, h_ref, al_ref, b_ref, out_ref):
    c = N_CLS
    e = al_ref[:, 0:1] + al_ref[:, 1:2]
    e = jnp.where(e >= 0.0, e, 0.2 * e)
    wself = jnp.exp(e)
    s = accA_ref[:, c:c + 1] + accB_ref[:, c:c + 1] + wself + 1e-16
    num = accA_ref[:, :c] + accB_ref[:, :c] + wself * h_ref[:, :c]
    g = num / s + b_ref[...]
    m = jnp.max(g, axis=1, keepdims=True)
    z = g - m
    out_ref[...] = z - jnp.log(jnp.sum(jnp.exp(z), axis=1, keepdims=True))


def _tc_final(accA, accB, h2e, al2, b2r):
    m, d = accA.shape
    c = N_CLS
    blk = 1024
    return pl.pallas_call(
        _final_body,
        grid=(m // blk,),
        in_specs=[
            pl.BlockSpec((blk, d), lambda i: (i, 0)),
            pl.BlockSpec((blk, d), lambda i: (i, 0)),
            pl.BlockSpec((blk, d), lambda i: (i, 0)),
            pl.BlockSpec((blk, 2), lambda i: (i, 0)),
            pl.BlockSpec((1, c), lambda i: (0, 0)),
        ],
        out_specs=pl.BlockSpec((blk, c), lambda i: (i, 0)),
        out_shape=jax.ShapeDtypeStruct((m, c), jnp.float32),
    )(accA, accB, h2e, al2, b2r)


KW = 1024
NWCHUNK = EPW // KW


def _sc_weights(with_s):
    mesh = plsc.VectorSubcoreMesh(core_axis_name="c", subcore_axis_name="s")

    out_type = [jax.ShapeDtypeStruct((E_PAD,), jnp.float32)]
    scratch = [
        pltpu.VMEM((N_PAD,), jnp.float32),
        pltpu.VMEM((N_PAD,), jnp.float32),
        pltpu.VMEM((KW,), jnp.int32),
        pltpu.VMEM((KW,), jnp.int32),
        pltpu.VMEM((KW,), jnp.int32),
        pltpu.VMEM((KW,), jnp.int32),
        pltpu.VMEM((KW,), jnp.float32),
        pltpu.VMEM((KW,), jnp.float32),
        pltpu.SemaphoreType.DMA,
        pltpu.SemaphoreType.DMA,
        pltpu.SemaphoreType.DMA,
        pltpu.SemaphoreType.DMA,
    ]
    if with_s:
        out_type.append(jax.ShapeDtypeStruct((NW, N_PAD), jnp.float32))
        scratch.insert(8, pltpu.VMEM((N_PAD,), jnp.float32))

    @functools.partial(
        pl.kernel,
        mesh=mesh,
        compiler_params=pltpu.CompilerParams(needs_layout_passes=False),
        out_type=out_type,
        scratch_types=scratch,
    )
    def k(src_hbm, dst_hbm, als_hbm, ald_hbm, *rest):
        if with_s:
            (w_hbm, s_hbm, als_v, ald_v, sidxA, sidxB, didxA, didxB,
             wstA, wstB, s_local, isemA, isemB, wsemA, wsemB) = rest
        else:
            (w_hbm, als_v, ald_v, sidxA, sidxB, didxA, didxB,
             wstA, wstB, isemA, isemB, wsemA, wsemB) = rest
        cid = lax.axis_index("c")
        sid = lax.axis_index("s")
        base = (cid * 16 + sid) * EPW
        zvec = jnp.zeros((16,), jnp.float32)

        pltpu.sync_copy(als_hbm, als_v)
        pltpu.sync_copy(ald_hbm, ald_v)
        if with_s:
            def _zs(i, _):
                s_local[pl.ds(i * 16, 16)] = zvec
                return 0
            lax.fori_loop(0, N_PAD // 16, _zs, 0)

        def _drain_idx(sem):
            pltpu.make_async_copy(src_hbm.at[pl.ds(0, KW)], sidxA, sem).wait()

        def _drain_w(sem):
            pltpu.make_async_copy(w_hbm.at[pl.ds(0, KW)], wstA, sem).wait()

        def _issue_idx(i, sidx, didx, isem):
            off = base + jnp.minimum(i, NWCHUNK - 1) * KW
            pltpu.async_copy(src_hbm.at[pl.ds(off, KW)], sidx, isem)
            pltpu.async_copy(dst_hbm.at[pl.ds(off, KW)], didx, isem)

        def _compute(i, sidx, didx, wst, wsem):
            def _grp(j, _):
                sv = sidx[pl.ds(j * 16, 16)]
                dv = didx[pl.ds(j * 16, 16)]
                e = plsc.load_gather(als_v, [sv]) + plsc.load_gather(ald_v, [dv])
                e = jnp.where(e >= 0.0, e, 0.2 * e)
                w = jnp.exp(e)
                wst[pl.ds(j * 16, 16)] = w
                if with_s:
                    plsc.addupdate_scatter(s_local, [dv], w)
                return 0
            lax.fori_loop(0, KW // 16, _grp, 0)
            pltpu.async_copy(wst, w_hbm.at[pl.ds(base + i * KW, KW)], wsem)

        def _half(i, sidxX, didxX, isemX, wsemX, sidxY, didxY, isemY, wstY, wsemY):
            @pl.when(i > 1)
            def _():
                _drain_w(wsemX)
            _issue_idx(i, sidxX, didxX, isemX)
            _drain_idx(isemY)
            _drain_idx(isemY)
            _compute(i - 1, sidxY, didxY, wstY, wsemY)

        _issue_idx(0, sidxA, didxA, isemA)

        def _pair(tt, _):
            i = 2 * tt + 1
            _half(i, sidxB, didxB, isemB, wsemB, sidxA, didxA, isemA, wstA, wsemA)
            _half(i + 1, sidxA, didxA, isemA, wsemA, sidxB, didxB, isemB, wstB, wsemB)
            return 0
        lax.fori_loop(0, NWCHUNK // 2, _pair, 0)

        _drain_idx(isemA)
        _drain_idx(isemA)
        _drain_w(wsemB)
        if with_s:
            pltpu.sync_copy(s_local, s_hbm.at[cid * 16 + sid])

    return k


def _sc_rows():
    mesh = plsc.VectorSubcoreMesh(core_axis_name="c", subcore_axis_name="s")
    NS = 4

    @functools.partial(
        pl.kernel,
        mesh=mesh,
        compiler_params=pltpu.CompilerParams(needs_layout_passes=False),
        out_type=jax.ShapeDtypeStruct((2, N_PAD, D), jnp.float32),
        scratch_types=(
            [pltpu.VMEM((KR,), jnp.int32) for _ in range(NS)] +
            [pltpu.VMEM((KR,), jnp.int32) for _ in range(NS)] +
            [pltpu.VMEM((KR,), jnp.float32) for _ in range(NS)] +
            [pltpu.VMEM((KR, D), jnp.float32) for _ in range(NS)] +
            [pltpu.VMEM_SHARED((N_PAD, D), jnp.float32)] +
            [pltpu.SemaphoreType.DMA for _ in range(3 * NS)]
        ),
    )
    def k(src_hbm, dst_hbm, w_hbm, h_hbm, out_hbm, *rest):
        sidx = rest[0:NS]
        didx = rest[NS:2 * NS]
        wbuf = rest[2 * NS:3 * NS]
        gbuf = rest[3 * NS:4 * NS]
        acc = rest[4 * NS]
        isem = rest[4 * NS + 1:4 * NS + 1 + NS]
        gsem = rest[4 * NS + 1 + NS:4 * NS + 1 + 2 * NS]
        ssem = rest[4 * NS + 1 + 2 * NS:4 * NS + 1 + 3 * NS]
        cid = lax.axis_index("c")
        sid = lax.axis_index("s")
        base = (cid * 16 + sid) * EPW
        zvec = jnp.zeros((16,), jnp.float32)

        def _zrow(i, _):
            for j in range(D // 16):
                gbuf[0][i, pl.ds(j * 16, 16)] = zvec
            return 0
        lax.fori_loop(0, KR, _zrow, 0)
        for t in range(ROWS_PER_TILE // KR):
            pltpu.sync_copy(gbuf[0],
                            acc.at[pl.ds(sid * ROWS_PER_TILE + t * KR, KR)])
        plsc.subcore_barrier()

        def _drain_small(sem):
            pltpu.make_async_copy(src_hbm.at[pl.ds(0, KR)], sidx[0], sem).wait()

        def _drain_rows(sem):
            pltpu.make_async_copy(h_hbm.at[pl.ds(0, KR)], gbuf[0], sem).wait()

        def _issue_idx(i, z):
            off = base + jnp.minimum(i, NRCHUNK - 1) * KR
            pltpu.async_copy(src_hbm.at[pl.ds(off, KR)], sidx[z], isem[z])
            pltpu.async_copy(dst_hbm.at[pl.ds(off, KR)], didx[z], isem[z])
            pltpu.async_copy(w_hbm.at[pl.ds(off, KR)], wbuf[z], isem[z])

        def _issue_gather(z):
            for _ in range(3):
                _drain_small(isem[z])
            pltpu.async_copy(h_hbm.at[sidx[z]], gbuf[z], gsem[z])

        def _scale(z):
            def _srow(r, _):
                wv = plsc.load_gather(wbuf[z], [jnp.full((16,), r, jnp.int32)])
                for q in range(D // 16):
                    gbuf[z][r, pl.ds(q * 16, 16)] = \
                        gbuf[z][r, pl.ds(q * 16, 16)] * wv
                return 0
            lax.fori_loop(0, KR, _srow, 0)

        def _half(i, x):
            z = (x + 2) % NS
            _drain_rows(gsem[x])
            _scale(x)
            pltpu.async_copy(gbuf[x], acc.at[didx[x]], ssem[x], add=True)
            @pl.when(i + 2 > NS - 1)
            def _():
                _drain_rows(ssem[z])
            _issue_idx(i + 2, z)
            _issue_gather(z)

        _issue_idx(0, 0)
        _issue_idx(1, 1)
        _issue_gather(0)
        _issue_gather(1)

        def _quad(tt, _):
            i = NS * tt
            for x in range(NS):
                _half(i + x, x)
            return 0
        lax.fori_loop(0, NRCHUNK // NS, _quad, 0)

        _drain_rows(ssem[NS - 2])
        _drain_rows(ssem[NS - 1])
        _drain_rows(gsem[0])
        _drain_rows(gsem[1])

        plsc.subcore_barrier()
        pltpu.sync_copy(acc.at[pl.ds(sid * ROWS_PER_TILE, ROWS_PER_TILE)],
                        out_hbm.at[cid, pl.ds(sid * ROWS_PER_TILE, ROWS_PER_TILE)])

    return k


_sc_w_l1 = _sc_weights(True)
_sc_w_l2 = _sc_weights(False)
_sc_rows_k = _sc_rows()


def kernel(x, edge_index, W1, a_src1, a_dst1, b1, W2, a_src2, a_dst2, b2):
    xp = jnp.pad(x, ((0, N_PAD - N), (0, 0)))
    npad = E_PAD - E
    srcp = jnp.concatenate(
        [edge_index[0], jnp.arange(npad, dtype=jnp.int32) * 997 % N])
    dstp = jnp.concatenate(
        [edge_index[1], N + (jnp.arange(npad, dtype=jnp.int32) % (N_PAD - N))])

    V1 = jnp.stack([a_src1, a_dst1], axis=1)
    W2e = jnp.pad(W2, ((0, 0), (0, D - N_CLS)))
    A2e = jnp.pad(jnp.stack([a_src2, a_dst2], axis=1), ((0, D - N_CLS), (0, 0)))
    b1r = b1.reshape(1, -1)
    b2r = b2.reshape(1, -1)

    h1, al1 = _tc_linatt(xp, W1, V1)
    w1, s1 = _sc_w_l1(srcp, dstp, al1[:, 0], al1[:, 1])
    acc1 = _sc_rows_k(srcp, dstp, w1, h1)
    h2e, al2 = _tc_mid(acc1[0], acc1[1], s1, h1, al1, b1r, W2e, A2e)
    w2 = _sc_w_l2(srcp, dstp, al2[:, 0], al2[:, 1])
    acc2 = _sc_rows_k(srcp, dstp, w2[0], h2e)
    out = _tc_final(acc2[0], acc2[1], h2e, al2, b2r)
    return out[:N]


# speedup vs baseline: 49.8106x; 1.1816x over previous
import functools

import jax
import jax.numpy as jnp
from jax import lax
from jax.experimental import pallas as pl
from jax.experimental.pallas import tpu as pltpu
from jax.experimental.pallas import tpu_sc as plsc

N = 10000
E = 320000
D_HID = 128
N_CLS = 40

N_PAD = 10240
NW = 32
E_PAD = 327680
EPW = E_PAD // NW
KR = 80
NRCHUNK = EPW // KR
D = 128

ROWS_PER_TILE = N_PAD // 16


def _linatt_body(x_ref, w_ref, v_ref, h_ref, al_ref):
    h = jnp.dot(x_ref[...], w_ref[...], preferred_element_type=jnp.float32)
    h_ref[...] = h
    al_ref[...] = jnp.dot(h, v_ref[...], preferred_element_type=jnp.float32)


def _tc_linatt(xp, W, V):
    m, d = xp.shape
    dh = W.shape[1]
    blk = 1024
    return pl.pallas_call(
        _linatt_body,
        grid=(m // blk,),
        in_specs=[
            pl.BlockSpec((blk, d), lambda i: (i, 0)),
            pl.BlockSpec((d, dh), lambda i: (0, 0)),
            pl.BlockSpec((dh, 2), lambda i: (0, 0)),
        ],
        out_specs=[
            pl.BlockSpec((blk, dh), lambda i: (i, 0)),
            pl.BlockSpec((blk, 2), lambda i: (i, 0)),
        ],
        out_shape=[
            jax.ShapeDtypeStruct((m, dh), jnp.float32),
            jax.ShapeDtypeStruct((m, 2), jnp.float32),
        ],
    )(xp, W, V)


def _mid_body(accA_ref, accB_ref, s32_ref, h_ref, al_ref, b_ref, w2_ref, a2_ref,
              h2_ref, al2_ref):
    e = al_ref[:, 0:1] + al_ref[:, 1:2]
    e = jnp.where(e >= 0.0, e, 0.2 * e)
    wself = jnp.exp(e)
    s = jnp.sum(s32_ref[...], axis=0)[:, None] + wself + 1e-16
    num = accA_ref[...] + accB_ref[...] + wself * h_ref[...]
    g = num / s + b_ref[...]
    h2pre = jnp.maximum(g, 0.0)
    h2 = jnp.dot(h2pre, w2_ref[...], preferred_element_type=jnp.float32)
    al2_ref[...] = jnp.dot(h2, a2_ref[...], preferred_element_type=jnp.float32)
    col = lax.broadcasted_iota(jnp.int32, h2.shape, 1)
    h2_ref[...] = jnp.where(col == N_CLS, 1.0, h2)


def _tc_mid(accA, accB, s32, h1, al1, b1r, W2e, A2e):
    m, d = accA.shape
    blk = 1024
    return pl.pallas_call(
        _mid_body,
        grid=(m // blk,),
        in_specs=[
            pl.BlockSpec((blk, d), lambda i: (i, 0)),
            pl.BlockSpec((blk, d), lambda i: (i, 0)),
            pl.BlockSpec((NW, blk), lambda i: (0, i)),
            pl.BlockSpec((blk, d), lambda i: (i, 0)),
            pl.BlockSpec((blk, 2), lambda i: (i, 0)),
            pl.BlockSpec((1, d), lambda i: (0, 0)),
            pl.BlockSpec((d, d), lambda i: (0, 0)),
            pl.BlockSpec((d, 2), lambda i: (0, 0)),
        ],
        out_specs=[
            pl.BlockSpec((blk, d), lambda i: (i, 0)),
            pl.BlockSpec((blk, 2), lambda i: (i, 0)),
        ],
        out_shape=[
            jax.ShapeDtypeStruct((m, d), jnp.float32),
            jax.ShapeDtypeStruct((m, 2), jnp.float32),
        ],
    )(accA, accB, s32, h1, al1, b1r, W2e, A2e)


def _final_body(accA_ref, accB_ref, h_ref, al_ref, b_ref, out_ref):
    c = N_CLS
    e = al_ref[:, 0:1] + al_ref[:, 1:2]
    e = jnp.where(e >= 0.0, e, 0.2 * e)
    wself = jnp.exp(e)
    s = accA_ref[:, c:c + 1] + accB_ref[:, c:c + 1] + wself + 1e-16
    num = accA_ref[:, :c] + accB_ref[:, :c] + wself * h_ref[:, :c]
    g = num / s + b_ref[...]
    m = jnp.max(g, axis=1, keepdims=True)
    z = g - m
    out_ref[...] = z - jnp.log(jnp.sum(jnp.exp(z), axis=1, keepdims=True))


def _tc_final(accA, accB, h2e, al2, b2r):
    m, d = accA.shape
    c = N_CLS
    blk = 1024
    return pl.pallas_call(
        _final_body,
        grid=(m // blk,),
        in_specs=[
            pl.BlockSpec((blk, d), lambda i: (i, 0)),
            pl.BlockSpec((blk, d), lambda i: (i, 0)),
            pl.BlockSpec((blk, d), lambda i: (i, 0)),
            pl.BlockSpec((blk, 2), lambda i: (i, 0)),
            pl.BlockSpec((1, c), lambda i: (0, 0)),
        ],
        out_specs=pl.BlockSpec((blk, c), lambda i: (i, 0)),
        out_shape=jax.ShapeDtypeStruct((m, c), jnp.float32),
    )(accA, accB, h2e, al2, b2r)


KW = 1024
NWCHUNK = EPW // KW


def _sc_weights(with_s):
    mesh = plsc.VectorSubcoreMesh(core_axis_name="c", subcore_axis_name="s")

    out_type = [jax.ShapeDtypeStruct((E_PAD,), jnp.float32)]
    scratch = [
        pltpu.VMEM((N_PAD,), jnp.float32),
        pltpu.VMEM((N_PAD,), jnp.float32),
        pltpu.VMEM((KW,), jnp.int32),
        pltpu.VMEM((KW,), jnp.int32),
        pltpu.VMEM((KW,), jnp.int32),
        pltpu.VMEM((KW,), jnp.int32),
        pltpu.VMEM((KW,), jnp.float32),
        pltpu.VMEM((KW,), jnp.float32),
        pltpu.SemaphoreType.DMA,
        pltpu.SemaphoreType.DMA,
        pltpu.SemaphoreType.DMA,
        pltpu.SemaphoreType.DMA,
    ]
    if with_s:
        out_type.append(jax.ShapeDtypeStruct((NW, N_PAD), jnp.float32))
        scratch.insert(8, pltpu.VMEM((N_PAD,), jnp.float32))

    @functools.partial(
        pl.kernel,
        mesh=mesh,
        compiler_params=pltpu.CompilerParams(needs_layout_passes=False),
        out_type=out_type,
        scratch_types=scratch,
    )
    def k(src_hbm, dst_hbm, als_hbm, ald_hbm, *rest):
        if with_s:
            (w_hbm, s_hbm, als_v, ald_v, sidxA, sidxB, didxA, didxB,
             wstA, wstB, s_local, isemA, isemB, wsemA, wsemB) = rest
        else:
            (w_hbm, als_v, ald_v, sidxA, sidxB, didxA, didxB,
             wstA, wstB, isemA, isemB, wsemA, wsemB) = rest
        cid = lax.axis_index("c")
        sid = lax.axis_index("s")
        base = (cid * 16 + sid) * EPW
        zvec = jnp.zeros((16,), jnp.float32)

        pltpu.sync_copy(als_hbm, als_v)
        pltpu.sync_copy(ald_hbm, ald_v)
        if with_s:
            def _zs(i, _):
                s_local[pl.ds(i * 16, 16)] = zvec
                return 0
            lax.fori_loop(0, N_PAD // 16, _zs, 0)

        def _drain_idx(sem):
            pltpu.make_async_copy(src_hbm.at[pl.ds(0, KW)], sidxA, sem).wait()

        def _drain_w(sem):
            pltpu.make_async_copy(w_hbm.at[pl.ds(0, KW)], wstA, sem).wait()

        def _issue_idx(i, sidx, didx, isem):
            off = base + jnp.minimum(i, NWCHUNK - 1) * KW
            pltpu.async_copy(src_hbm.at[pl.ds(off, KW)], sidx, isem)
            pltpu.async_copy(dst_hbm.at[pl.ds(off, KW)], didx, isem)

        def _compute(i, sidx, didx, wst, wsem):
            @plsc.parallel_loop(0, KW // 16, unroll=4)
            def _grp(j):
                sv = sidx[pl.ds(j * 16, 16)]
                dv = didx[pl.ds(j * 16, 16)]
                e = plsc.load_gather(als_v, [sv]) + plsc.load_gather(ald_v, [dv])
                e = jnp.where(e >= 0.0, e, 0.2 * e)
                w = jnp.exp(e)
                wst[pl.ds(j * 16, 16)] = w
                if with_s:
                    plsc.addupdate_scatter(s_local, [dv], w)
            pltpu.async_copy(wst, w_hbm.at[pl.ds(base + i * KW, KW)], wsem)

        def _half(i, sidxX, didxX, isemX, wsemX, sidxY, didxY, isemY, wstY, wsemY):
            @pl.when(i > 1)
            def _():
                _drain_w(wsemX)
            _issue_idx(i, sidxX, didxX, isemX)
            _drain_idx(isemY)
            _drain_idx(isemY)
            _compute(i - 1, sidxY, didxY, wstY, wsemY)

        _issue_idx(0, sidxA, didxA, isemA)

        def _pair(tt, _):
            i = 2 * tt + 1
            _half(i, sidxB, didxB, isemB, wsemB, sidxA, didxA, isemA, wstA, wsemA)
            _half(i + 1, sidxA, didxA, isemA, wsemA, sidxB, didxB, isemB, wstB, wsemB)
            return 0
        lax.fori_loop(0, NWCHUNK // 2, _pair, 0)

        _drain_idx(isemA)
        _drain_idx(isemA)
        _drain_w(wsemB)
        if with_s:
            pltpu.sync_copy(s_local, s_hbm.at[cid * 16 + sid])

    return k


def _sc_rows():
    mesh = plsc.VectorSubcoreMesh(core_axis_name="c", subcore_axis_name="s")
    NS = 4

    @functools.partial(
        pl.kernel,
        mesh=mesh,
        compiler_params=pltpu.CompilerParams(needs_layout_passes=False),
        out_type=jax.ShapeDtypeStruct((2, N_PAD, D), jnp.float32),
        scratch_types=(
            [pltpu.VMEM((KR,), jnp.int32) for _ in range(NS)] +
            [pltpu.VMEM((KR,), jnp.int32) for _ in range(NS)] +
            [pltpu.VMEM((KR,), jnp.float32) for _ in range(NS)] +
            [pltpu.VMEM((KR, D), jnp.float32) for _ in range(NS)] +
            [pltpu.VMEM_SHARED((N_PAD, D), jnp.float32)] +
            [pltpu.SemaphoreType.DMA for _ in range(3 * NS)]
        ),
    )
    def k(src_hbm, dst_hbm, w_hbm, h_hbm, out_hbm, *rest):
        sidx = rest[0:NS]
        didx = rest[NS:2 * NS]
        wbuf = rest[2 * NS:3 * NS]
        gbuf = rest[3 * NS:4 * NS]
        acc = rest[4 * NS]
        isem = rest[4 * NS + 1:4 * NS + 1 + NS]
        gsem = rest[4 * NS + 1 + NS:4 * NS + 1 + 2 * NS]
        ssem = rest[4 * NS + 1 + 2 * NS:4 * NS + 1 + 3 * NS]
        cid = lax.axis_index("c")
        sid = lax.axis_index("s")
        base = (cid * 16 + sid) * EPW
        zvec = jnp.zeros((16,), jnp.float32)

        def _zrow(i, _):
            for j in range(D // 16):
                gbuf[0][i, pl.ds(j * 16, 16)] = zvec
            return 0
        lax.fori_loop(0, KR, _zrow, 0)
        for t in range(ROWS_PER_TILE // KR):
            pltpu.sync_copy(gbuf[0],
                            acc.at[pl.ds(sid * ROWS_PER_TILE + t * KR, KR)])
        plsc.subcore_barrier()

        def _drain_small(sem):
            pltpu.make_async_copy(src_hbm.at[pl.ds(0, KR)], sidx[0], sem).wait()

        def _drain_rows(sem):
            pltpu.make_async_copy(h_hbm.at[pl.ds(0, KR)], gbuf[0], sem).wait()

        def _issue_idx(i, z):
            off = base + jnp.minimum(i, NRCHUNK - 1) * KR
            pltpu.async_copy(src_hbm.at[pl.ds(off, KR)], sidx[z], isem[z])
            pltpu.async_copy(dst_hbm.at[pl.ds(off, KR)], didx[z], isem[z])
            pltpu.async_copy(w_hbm.at[pl.ds(off, KR)], wbuf[z], isem[z])

        def _issue_gather(z):
            for _ in range(3):
                _drain_small(isem[z])
            pltpu.async_copy(h_hbm.at[sidx[z]], gbuf[z], gsem[z])

        def _scale(z):
            @plsc.parallel_loop(0, KR, unroll=4)
            def _srow(r):
                wv = plsc.load_gather(wbuf[z], [jnp.full((16,), r, jnp.int32)])
                for q in range(D // 16):
                    gbuf[z][r, pl.ds(q * 16, 16)] = \
                        gbuf[z][r, pl.ds(q * 16, 16)] * wv

        def _half(i, x):
            z = (x + 2) % NS
            _drain_rows(gsem[x])
            _scale(x)
            pltpu.async_copy(gbuf[x], acc.at[didx[x]], ssem[x], add=True)
            @pl.when(i + 2 > NS - 1)
            def _():
                _drain_rows(ssem[z])
            _issue_idx(i + 2, z)
            _issue_gather(z)

        _issue_idx(0, 0)
        _issue_idx(1, 1)
        _issue_gather(0)
        _issue_gather(1)

        def _quad(tt, _):
            i = NS * tt
            for x in range(NS):
                _half(i + x, x)
            return 0
        lax.fori_loop(0, NRCHUNK // NS, _quad, 0)

        _drain_rows(ssem[NS - 2])
        _drain_rows(ssem[NS - 1])
        _drain_rows(gsem[0])
        _drain_rows(gsem[1])

        plsc.subcore_barrier()
        pltpu.sync_copy(acc.at[pl.ds(sid * ROWS_PER_TILE, ROWS_PER_TILE)],
                        out_hbm.at[cid, pl.ds(sid * ROWS_PER_TILE, ROWS_PER_TILE)])

    return k


_sc_w_l1 = _sc_weights(True)
_sc_w_l2 = _sc_weights(False)
_sc_rows_k = _sc_rows()


def kernel(x, edge_index, W1, a_src1, a_dst1, b1, W2, a_src2, a_dst2, b2):
    xp = jnp.pad(x, ((0, N_PAD - N), (0, 0)))
    npad = E_PAD - E
    srcp = jnp.concatenate(
        [edge_index[0], jnp.arange(npad, dtype=jnp.int32) * 997 % N])
    dstp = jnp.concatenate(
        [edge_index[1], N + (jnp.arange(npad, dtype=jnp.int32) % (N_PAD - N))])

    V1 = jnp.stack([a_src1, a_dst1], axis=1)
    W2e = jnp.pad(W2, ((0, 0), (0, D - N_CLS)))
    A2e = jnp.pad(jnp.stack([a_src2, a_dst2], axis=1), ((0, D - N_CLS), (0, 0)))
    b1r = b1.reshape(1, -1)
    b2r = b2.reshape(1, -1)

    h1, al1 = _tc_linatt(xp, W1, V1)
    w1, s1 = _sc_w_l1(srcp, dstp, al1[:, 0], al1[:, 1])
    acc1 = _sc_rows_k(srcp, dstp, w1, h1)
    h2e, al2 = _tc_mid(acc1[0], acc1[1], s1, h1, al1, b1r, W2e, A2e)
    w2 = _sc_w_l2(srcp, dstp, al2[:, 0], al2[:, 1])
    acc2 = _sc_rows_k(srcp, dstp, w2[0], h2e)
    out = _tc_final(acc2[0], acc2[1], h2e, al2, b2r)
    return out[:N]
